# baseline (speedup 1.0000x reference)
import functools

import jax
import jax.numpy as jnp
from jax.experimental import pallas as pl
from jax.experimental.pallas import tpu as pltpu

_TILE = 2048
_NBUF = 4
_R = 2
_PD = 12


def _fc_kernel(nfull, tail, x_ref, w_ref, b_ref, o_ref,
               acc_ref, wbuf_ref, sems, wsems):
    j = pl.program_id(0)
    nstep = pl.num_programs(0)
    slot = jax.lax.rem(j, _NBUF)
    wslot = jax.lax.rem(j, _PD)

    def out_copies(step, s, rows):
        half = (rows // 2) // 8 * 8
        sizes = (half, rows - half)
        offs = (0, half)
        return [
            pltpu.make_async_copy(
                acc_ref.at[s, pl.ds(offs[r], sizes[r]), :],
                o_ref.at[pl.ds(step * _TILE + offs[r], sizes[r]), :],
                sems.at[s, r],
            )
            for r in range(_R)
        ]

    def w_copy(step, rows):
        s = jax.lax.rem(step, _PD)
        return pltpu.make_async_copy(
            w_ref.at[pl.ds(step * _TILE, rows), :],
            wbuf_ref.at[s, pl.ds(0, rows), :],
            wsems.at[s],
        )

    @pl.when(j == 0)
    def _prologue():
        for k in range(min(_PD, nstep)):
            w_copy(k, _TILE if k < nfull else tail).start()

    @pl.when(j >= _NBUF)
    def _wait_prev():
        for c in out_copies(j - _NBUF, slot, _TILE):
            c.wait()

    @pl.when(j < nfull)
    def _wait_w_full():
        w_copy(j, _TILE).wait()

    if tail:
        @pl.when(j == nfull)
        def _wait_w_tail():
            w_copy(j, tail).wait()

    xb = x_ref[...]
    wb = wbuf_ref[wslot].astype(jnp.bfloat16)
    onehot = (jax.lax.broadcasted_iota(jnp.int32, (b_ref.shape[1], 1), 0)
              == j).astype(jnp.bfloat16)
    bcol = jax.lax.dot_general(
        b_ref[...], onehot, (((1,), (0,)), ((), ())),
        preferred_element_type=jnp.float32,
    )
    acc_ref[slot] = jax.lax.dot_general(
        wb, xb, (((1,), (1,)), ((), ())),
        preferred_element_type=jnp.float32,
    ) + bcol

    @pl.when(j < nfull)
    def _start_full():
        for r, c in enumerate(out_copies(j, slot, _TILE)):
            c.start(priority=r % 2)

    if tail:
        @pl.when(j == nfull)
        def _start_tail():
            for r, c in enumerate(out_copies(j, slot, tail)):
                c.start(priority=r % 2)

    nxt = j + _PD
    @pl.when(nxt < nfull)
    def _prefetch_full():
        w_copy(nxt, _TILE).start()

    if tail:
        @pl.when(nxt == nfull)
        def _prefetch_tail():
            w_copy(nxt, tail).start()

    @pl.when(j == nstep - 1)
    def _drain():
        for d in range(1, _NBUF):
            pj = j - d
            for c in out_copies(pj, jax.lax.rem(pj, _NBUF), _TILE):
                c.wait()
        for c in out_copies(j, slot, tail if tail else _TILE):
            c.wait()


def kernel(x, W, b):
    batch, in_dim = x.shape
    out_dim = W.shape[0]
    nfull = out_dim // _TILE
    tail = out_dim - nfull * _TILE
    nstep = nfull + (1 if tail else 0)
    bpad = jnp.pad(b, (0, nstep * _TILE - out_dim))
    blanes = max(128, -(-nstep // 128) * 128)
    b2 = jnp.pad(bpad.reshape(nstep, _TILE).T,
                 ((0, 0), (0, blanes - nstep))).astype(jnp.bfloat16)
    xbf = x.astype(jnp.bfloat16)
    yT = pl.pallas_call(
        functools.partial(_fc_kernel, nfull, tail),
        grid=(nstep,),
        in_specs=[
            pl.BlockSpec((batch, in_dim), lambda j: (0, 0),
                         pipeline_mode=pl.Buffered(buffer_count=1)),
            pl.BlockSpec(memory_space=pl.ANY),
            pl.BlockSpec((_TILE, blanes), lambda j: (0, 0),
                         pipeline_mode=pl.Buffered(buffer_count=1)),
        ],
        out_specs=pl.BlockSpec(memory_space=pl.ANY),
        out_shape=jax.ShapeDtypeStruct((out_dim, batch), jnp.float32),
        scratch_shapes=[
            pltpu.VMEM((_NBUF, _TILE, batch), jnp.float32),
            pltpu.VMEM((_PD, _TILE, in_dim), jnp.float32),
            pltpu.SemaphoreType.DMA((_NBUF, _R)),
            pltpu.SemaphoreType.DMA((_PD,)),
        ],
        compiler_params=pltpu.CompilerParams(
            dimension_semantics=("arbitrary",),
        ),
    )(xbf, W, b2)
    return yT.T


# speedup vs baseline: 1.0056x; 1.0056x over previous
import functools

import jax
import jax.numpy as jnp
from jax.experimental import pallas as pl
from jax.experimental.pallas import tpu as pltpu

_TILE = 2560
_NBUF = 4
_R = 2
_PD = 8


def _fc_kernel(nfull, tail, x_ref, w_ref, b_ref, o_ref,
               acc_ref, wbuf_ref, sems, wsems):
    j = pl.program_id(0)
    nstep = pl.num_programs(0)
    slot = jax.lax.rem(j, _NBUF)
    wslot = jax.lax.rem(j, _PD)

    def out_copies(step, s, rows):
        half = (rows // 2) // 8 * 8
        sizes = (half, rows - half)
        offs = (0, half)
        return [
            pltpu.make_async_copy(
                acc_ref.at[s, pl.ds(offs[r], sizes[r]), :],
                o_ref.at[pl.ds(step * _TILE + offs[r], sizes[r]), :],
                sems.at[s, r],
            )
            for r in range(_R)
        ]

    def w_copy(step, rows):
        s = jax.lax.rem(step, _PD)
        return pltpu.make_async_copy(
            w_ref.at[pl.ds(step * _TILE, rows), :],
            wbuf_ref.at[s, pl.ds(0, rows), :],
            wsems.at[s],
        )

    @pl.when(j == 0)
    def _prologue():
        for k in range(min(_PD, nstep)):
            w_copy(k, _TILE if k < nfull else tail).start()

    @pl.when(j >= _NBUF)
    def _wait_prev():
        for c in out_copies(j - _NBUF, slot, _TILE):
            c.wait()

    @pl.when(j < nfull)
    def _wait_w_full():
        w_copy(j, _TILE).wait()

    if tail:
        @pl.when(j == nfull)
        def _wait_w_tail():
            w_copy(j, tail).wait()

    xb = x_ref[...].astype(jnp.bfloat16)
    wb = wbuf_ref[wslot].astype(jnp.bfloat16)
    onehot = (jax.lax.broadcasted_iota(jnp.int32, (b_ref.shape[1], 1), 0)
              == j).astype(jnp.float32)
    bcol = jax.lax.dot_general(
        b_ref[...], onehot, (((1,), (0,)), ((), ())),
        preferred_element_type=jnp.float32,
    )
    acc_ref[slot] = jax.lax.dot_general(
        wb, xb, (((1,), (1,)), ((), ())),
        preferred_element_type=jnp.float32,
    ) + bcol

    @pl.when(j < nfull)
    def _start_full():
        for r, c in enumerate(out_copies(j, slot, _TILE)):
            c.start(priority=r % 2)

    if tail:
        @pl.when(j == nfull)
        def _start_tail():
            for r, c in enumerate(out_copies(j, slot, tail)):
                c.start(priority=r % 2)

    nxt = j + _PD
    @pl.when(nxt < nfull)
    def _prefetch_full():
        w_copy(nxt, _TILE).start()

    if tail:
        @pl.when(nxt == nfull)
        def _prefetch_tail():
            w_copy(nxt, tail).start()

    @pl.when(j == nstep - 1)
    def _drain():
        for d in range(1, _NBUF):
            pj = j - d
            for c in out_copies(pj, jax.lax.rem(pj, _NBUF), _TILE):
                c.wait()
        for c in out_copies(j, slot, tail if tail else _TILE):
            c.wait()


def kernel(x, W, b):
    batch, in_dim = x.shape
    out_dim = W.shape[0]
    nfull = out_dim // _TILE
    tail = out_dim - nfull * _TILE
    nstep = nfull + (1 if tail else 0)
    bpad = jnp.pad(b, (0, nstep * _TILE - out_dim))
    blanes = max(128, -(-nstep // 128) * 128)
    b2 = jnp.pad(bpad.reshape(nstep, _TILE).T, ((0, 0), (0, blanes - nstep)))
    yT = pl.pallas_call(
        functools.partial(_fc_kernel, nfull, tail),
        grid=(nstep,),
        in_specs=[
            pl.BlockSpec((batch, in_dim), lambda j: (0, 0),
                         pipeline_mode=pl.Buffered(buffer_count=1)),
            pl.BlockSpec(memory_space=pl.ANY),
            pl.BlockSpec((_TILE, blanes), lambda j: (0, 0),
                         pipeline_mode=pl.Buffered(buffer_count=1)),
        ],
        out_specs=pl.BlockSpec(memory_space=pl.ANY),
        out_shape=jax.ShapeDtypeStruct((out_dim, batch), jnp.float32),
        scratch_shapes=[
            pltpu.VMEM((_NBUF, _TILE, batch), jnp.float32),
            pltpu.VMEM((_PD, _TILE, in_dim), jnp.float32),
            pltpu.SemaphoreType.DMA((_NBUF, _R)),
            pltpu.SemaphoreType.DMA((_PD,)),
        ],
        compiler_params=pltpu.CompilerParams(
            dimension_semantics=("arbitrary",),
        ),
    )(x, W, b2)
    return yT.T
